# baseline (speedup 1.0000x reference)
import functools

import jax
import jax.numpy as jnp
from jax import lax
from jax.experimental import pallas as pl
from jax.experimental.pallas import tpu as pltpu, tpu_sc as plsc

N = 10000
E = 320000
D = 128
H = 64
OUT = 64
G = 64

NC, NS = 2, 16
NW = NC * NS
NPAD = 10240
CW = 16

NSUB, SUBW = 1, 250
CHUNK = NSUB * SUBW
EPT = E // NW
NCHUNK = EPT // CHUNK
RPT = NPAD // NS
FRT = N // NS

RT = 1000
NBLK = N // RT


def _edge_agg_body(with_counts, *refs):
    if with_counts:
        (p_hbm, er, z64, z16, ones_h,
         agg_out, cnt_out, sidx, didx, rows, ones_v, acc, cacc,
         gsem, ssem) = refs
    else:
        (p_hbm, er, z64,
         agg_out, sidx, didx, rows, acc, gsem, ssem) = refs

    cid = lax.axis_index("c")
    sid = lax.axis_index("s")
    wid = sid * NC + cid

    rsl = pl.ds(sid * RPT, RPT)
    pltpu.sync_copy(z64, acc.at[rsl])
    pltpu.sync_copy(er.at[0, wid], sidx)
    pltpu.sync_copy(er.at[1, wid], didx)
    if with_counts:
        pltpu.sync_copy(z16, cacc.at[rsl])
        pltpu.sync_copy(ones_h, ones_v)
    plsc.subcore_barrier()

    def fire_gathers(s, b):
        for j in range(NSUB):
            pltpu.make_async_copy(p_hbm.at[sidx.at[s, j]], rows.at[b, j],
                                  gsem).start()

    def wait_gathers(b):
        for j in range(NSUB):
            pltpu.make_async_copy(p_hbm.at[sidx.at[0, j]], rows.at[b, j],
                                  gsem).wait()

    def fire_scatters(s, b):
        for j in range(NSUB):
            pltpu.make_async_copy(rows.at[b, j], acc.at[didx.at[s, j]],
                                  ssem).start(add=True)
        if with_counts:
            for j in range(NSUB):
                pltpu.make_async_copy(ones_v, cacc.at[didx.at[s, j]],
                                      ssem).start(add=True)

    def drain_scatters(s, b):
        for j in range(NSUB):
            pltpu.make_async_copy(rows.at[b, j], acc.at[didx.at[s, j]],
                                  ssem).wait()
        if with_counts:
            for j in range(NSUB):
                pltpu.make_async_copy(ones_v, cacc.at[didx.at[s, j]],
                                      ssem).wait()

    fire_gathers(0, 0)

    def chunk(s, carry):
        b = lax.rem(s, 2)
        nb = 1 - b
        wait_gathers(b)
        fire_scatters(s, b)

        @pl.when(s < NCHUNK - 1)
        def _():
            @pl.when(s >= 1)
            def _():
                drain_scatters(s - 1, nb)
            fire_gathers(s + 1, nb)

        return carry

    lax.fori_loop(0, NCHUNK, chunk, 0)
    drain_scatters(NCHUNK - 2, (NCHUNK - 2) % 2)
    drain_scatters(NCHUNK - 1, (NCHUNK - 1) % 2)
    plsc.subcore_barrier()

    fsl = pl.ds(sid * FRT, FRT)
    pltpu.sync_copy(acc.at[fsl], agg_out.at[cid, fsl])
    if with_counts:
        pltpu.sync_copy(cacc.at[fsl], cnt_out.at[cid, fsl])


def _make_edge_agg(with_counts):
    mesh = plsc.VectorSubcoreMesh(core_axis_name="c", subcore_axis_name="s",
                                  num_cores=NC, num_subcores=NS)
    out_type = [jax.ShapeDtypeStruct((NC, N, H), jnp.float32)]
    scratch = [
        pltpu.VMEM((NCHUNK, NSUB, SUBW), jnp.int32),
        pltpu.VMEM((NCHUNK, NSUB, SUBW), jnp.int32),
        pltpu.VMEM((2, NSUB, SUBW, H), jnp.float32),
    ]
    if with_counts:
        out_type.append(jax.ShapeDtypeStruct((NC, N, CW), jnp.float32))
        scratch.append(pltpu.VMEM((SUBW, CW), jnp.float32))
    scratch.append(pltpu.VMEM_SHARED((NPAD, H), jnp.float32))
    if with_counts:
        scratch.append(pltpu.VMEM_SHARED((NPAD, CW), jnp.float32))
    scratch.append(pltpu.SemaphoreType.DMA)
    scratch.append(pltpu.SemaphoreType.DMA)
    return pl.kernel(
        functools.partial(_edge_agg_body, with_counts),
        out_type=out_type,
        mesh=mesh,
        scratch_types=scratch,
        compiler_params=pltpu.CompilerParams(use_tc_tiling_on_sc=False),
        name="edge_agg_cnt" if with_counts else "edge_agg",
    )


def _pre_body(x_ref, w1l_ref, w1r_ref, b1_ref, p1_ref, r1_ref):
    xb = x_ref[...]
    dn = (((1,), (1,)), ((), ()))
    p1_ref[...] = lax.dot_general(xb, w1l_ref[...], dn,
                                  preferred_element_type=jnp.float32)
    r1_ref[...] = lax.dot_general(xb, w1r_ref[...], dn,
                                  preferred_element_type=jnp.float32) + b1_ref[0:1, :]


def _mid_body(a0_ref, a1_ref, c0_ref, c1_ref, r1_ref, w2l_ref, w2r_ref,
              b2_ref, p2_ref, q2_ref):
    cnt = jnp.maximum(c0_ref[0][:, 0:1] + c1_ref[0][:, 0:1], 1.0)
    h = jnp.maximum((a0_ref[0] + a1_ref[0]) / cnt + r1_ref[...], 0.0)
    dn = (((1,), (1,)), ((), ()))
    p2_ref[...] = lax.dot_general(h, w2l_ref[...], dn,
                                  preferred_element_type=jnp.float32)
    q2_ref[...] = lax.dot_general(h, w2r_ref[...], dn,
                                  preferred_element_type=jnp.float32) + b2_ref[0:1, :]


def _pool_body(a0_ref, a1_ref, c0_ref, c1_ref, q2_ref, b_ref, out_ref):
    i = pl.program_id(0)
    cnt = jnp.maximum(c0_ref[0][:, 0:1] + c1_ref[0][:, 0:1], 1.0)
    h2 = (a0_ref[0] + a1_ref[0]) / cnt + q2_ref[...]
    bidx = b_ref[0, 0, :]
    onehot = (bidx[:, None] == lax.broadcasted_iota(jnp.int32, (1, G), 1)
              ).astype(jnp.float32)
    h2a = jnp.concatenate([h2, jnp.ones((RT, 64), jnp.float32)], axis=1)
    contrib = lax.dot_general(onehot, h2a, (((0,), (0,)), ((), ())),
                              preferred_element_type=jnp.float32)

    @pl.when(i == 0)
    def _():
        out_ref[...] = contrib

    @pl.when(i > 0)
    def _():
        out_ref[...] += contrib

    @pl.when(i == NBLK - 1)
    def _():
        o = out_ref[...]
        out_ref[...] = o / jnp.maximum(o[:, 64:65], 1.0)


_full = lambda i: (0, 0)


def kernel(x, edge_index, batch_index, W1l, W1r, b1, W2l, W2r, b2):
    f32 = jnp.float32
    er = edge_index.reshape(2, NW, NCHUNK, NSUB, SUBW)
    bidx = batch_index.reshape(NBLK, 1, RT)
    z64 = jnp.zeros((RPT, H), f32)
    z16 = jnp.zeros((RPT, CW), f32)
    ones_h = jnp.ones((SUBW, CW), f32)
    b1t = jnp.broadcast_to(b1, (8, H))
    b2t = jnp.broadcast_to(b2, (8, OUT))

    p1, r1 = pl.pallas_call(
        _pre_body,
        grid=(NBLK,),
        in_specs=[
            pl.BlockSpec((RT, D), lambda i: (i, 0)),
            pl.BlockSpec((H, D), _full),
            pl.BlockSpec((H, D), _full),
            pl.BlockSpec((8, H), _full),
        ],
        out_specs=[pl.BlockSpec((RT, H), lambda i: (i, 0))] * 2,
        out_shape=[jax.ShapeDtypeStruct((N, H), f32)] * 2,
    )(x, W1l, W1r, b1t)

    agg1, cnt = _make_edge_agg(True)(p1, er, z64, z16, ones_h)

    a3 = pl.BlockSpec((1, RT, H), lambda i: (0, i, 0))
    a3b = pl.BlockSpec((1, RT, H), lambda i: (1, i, 0))
    c3 = pl.BlockSpec((1, RT, CW), lambda i: (0, i, 0))
    c3b = pl.BlockSpec((1, RT, CW), lambda i: (1, i, 0))
    p2, q2 = pl.pallas_call(
        _mid_body,
        grid=(NBLK,),
        in_specs=[
            a3, a3b, c3, c3b,
            pl.BlockSpec((RT, H), lambda i: (i, 0)),
            pl.BlockSpec((OUT, H), _full),
            pl.BlockSpec((OUT, H), _full),
            pl.BlockSpec((8, OUT), _full),
        ],
        out_specs=[pl.BlockSpec((RT, OUT), lambda i: (i, 0))] * 2,
        out_shape=[jax.ShapeDtypeStruct((N, OUT), f32)] * 2,
    )(agg1, agg1, cnt, cnt, r1, W2l, W2r, b2t)

    agg2, = _make_edge_agg(False)(p2, er, z64)

    pooled = pl.pallas_call(
        _pool_body,
        grid=(NBLK,),
        in_specs=[
            a3, a3b, c3, c3b,
            pl.BlockSpec((RT, OUT), lambda i: (i, 0)),
            pl.BlockSpec((1, 1, RT), lambda i: (i, 0, 0)),
        ],
        out_specs=pl.BlockSpec((G, 128), _full),
        out_shape=jax.ShapeDtypeStruct((G, 128), f32),
    )(agg2, agg2, cnt, cnt, q2, bidx)

    return pooled[:, :OUT]


# speedup vs baseline: 1.1223x; 1.1223x over previous
import functools

import jax
import jax.numpy as jnp
from jax import lax
from jax.experimental import pallas as pl
from jax.experimental.pallas import tpu as pltpu, tpu_sc as plsc

N = 10000
E = 320000
D = 128
H = 64
OUT = 64
G = 64

NC, NS = 2, 16
NW = NC * NS
NPAD = 10240
CW = 16

NSUB, SUBW = 1, 250
CHUNK = NSUB * SUBW
EPT = E // NW
NCHUNK = EPT // CHUNK
DEPTH = 3
RPT = NPAD // NS
FRT = N // NS

RT = 1000
NBLK = N // RT


def _edge_agg_body(with_counts, *refs):
    if with_counts:
        (p_hbm, er, z64, z16, ones_h,
         agg_out, cnt_out, sidx, didx, rows, ones_v, acc, cacc,
         gsem, ssem) = refs
    else:
        (p_hbm, er, z64,
         agg_out, sidx, didx, rows, acc, gsem, ssem) = refs

    cid = lax.axis_index("c")
    sid = lax.axis_index("s")
    wid = sid * NC + cid

    rsl = pl.ds(sid * RPT, RPT)
    pltpu.sync_copy(z64, acc.at[rsl])
    pltpu.sync_copy(er.at[0, wid], sidx)
    pltpu.sync_copy(er.at[1, wid], didx)
    if with_counts:
        pltpu.sync_copy(z16, cacc.at[rsl])
        pltpu.sync_copy(ones_h, ones_v)
    plsc.subcore_barrier()

    def fire_gathers(s, b):
        for j in range(NSUB):
            pltpu.make_async_copy(p_hbm.at[sidx.at[s, j]], rows.at[b, j],
                                  gsem).start()

    def wait_gathers(b):
        for j in range(NSUB):
            pltpu.make_async_copy(p_hbm.at[sidx.at[0, j]], rows.at[b, j],
                                  gsem).wait()

    def fire_scatters(s, b):
        for j in range(NSUB):
            pltpu.make_async_copy(rows.at[b, j], acc.at[didx.at[s, j]],
                                  ssem).start(add=True)
        if with_counts:
            for j in range(NSUB):
                pltpu.make_async_copy(ones_v, cacc.at[didx.at[s, j]],
                                      ssem).start(add=True)

    def drain_scatters(s, b):
        for j in range(NSUB):
            pltpu.make_async_copy(rows.at[b, j], acc.at[didx.at[s, j]],
                                  ssem).wait()
        if with_counts:
            for j in range(NSUB):
                pltpu.make_async_copy(ones_v, cacc.at[didx.at[s, j]],
                                      ssem).wait()

    for k in range(DEPTH - 1):
        fire_gathers(k, k)

    def chunk(s, carry):
        b = lax.rem(s, DEPTH)
        pf = s + DEPTH - 1

        @pl.when(pf < NCHUNK)
        def _():
            @pl.when(s >= 1)
            def _():
                drain_scatters(s - 1, lax.rem(s - 1, DEPTH))
            fire_gathers(pf, lax.rem(pf, DEPTH))

        wait_gathers(b)
        fire_scatters(s, b)
        return carry

    lax.fori_loop(0, NCHUNK, chunk, 0)
    for s in range(NCHUNK - DEPTH, NCHUNK):
        drain_scatters(s, s % DEPTH)
    plsc.subcore_barrier()

    fsl = pl.ds(sid * FRT, FRT)
    pltpu.sync_copy(acc.at[fsl], agg_out.at[cid, fsl])
    if with_counts:
        pltpu.sync_copy(cacc.at[fsl], cnt_out.at[cid, fsl])


def _make_edge_agg(with_counts):
    mesh = plsc.VectorSubcoreMesh(core_axis_name="c", subcore_axis_name="s",
                                  num_cores=NC, num_subcores=NS)
    out_type = [jax.ShapeDtypeStruct((NC, N, H), jnp.float32)]
    scratch = [
        pltpu.VMEM((NCHUNK, NSUB, SUBW), jnp.int32),
        pltpu.VMEM((NCHUNK, NSUB, SUBW), jnp.int32),
        pltpu.VMEM((DEPTH, NSUB, SUBW, H), jnp.float32),
    ]
    if with_counts:
        out_type.append(jax.ShapeDtypeStruct((NC, N, CW), jnp.float32))
        scratch.append(pltpu.VMEM((SUBW, CW), jnp.float32))
    scratch.append(pltpu.VMEM_SHARED((NPAD, H), jnp.float32))
    if with_counts:
        scratch.append(pltpu.VMEM_SHARED((NPAD, CW), jnp.float32))
    scratch.append(pltpu.SemaphoreType.DMA)
    scratch.append(pltpu.SemaphoreType.DMA)
    return pl.kernel(
        functools.partial(_edge_agg_body, with_counts),
        out_type=out_type,
        mesh=mesh,
        scratch_types=scratch,
        compiler_params=pltpu.CompilerParams(use_tc_tiling_on_sc=False),
        name="edge_agg_cnt" if with_counts else "edge_agg",
    )


def _pre_body(x_ref, w1l_ref, w1r_ref, b1_ref, p1_ref, r1_ref):
    xb = x_ref[...]
    dn = (((1,), (1,)), ((), ()))
    p1_ref[...] = lax.dot_general(xb, w1l_ref[...], dn,
                                  preferred_element_type=jnp.float32)
    r1_ref[...] = lax.dot_general(xb, w1r_ref[...], dn,
                                  preferred_element_type=jnp.float32) + b1_ref[0:1, :]


def _mid_body(a0_ref, a1_ref, c0_ref, c1_ref, r1_ref, w2l_ref, w2r_ref,
              b2_ref, p2_ref, q2_ref):
    cnt = jnp.maximum(c0_ref[0][:, 0:1] + c1_ref[0][:, 0:1], 1.0)
    h = jnp.maximum((a0_ref[0] + a1_ref[0]) / cnt + r1_ref[...], 0.0)
    dn = (((1,), (1,)), ((), ()))
    p2_ref[...] = lax.dot_general(h, w2l_ref[...], dn,
                                  preferred_element_type=jnp.float32)
    q2_ref[...] = lax.dot_general(h, w2r_ref[...], dn,
                                  preferred_element_type=jnp.float32) + b2_ref[0:1, :]


def _pool_body(a0_ref, a1_ref, c0_ref, c1_ref, q2_ref, b_ref, out_ref):
    i = pl.program_id(0)
    cnt = jnp.maximum(c0_ref[0][:, 0:1] + c1_ref[0][:, 0:1], 1.0)
    h2 = (a0_ref[0] + a1_ref[0]) / cnt + q2_ref[...]
    bidx = b_ref[0, 0, :]
    onehot = (bidx[:, None] == lax.broadcasted_iota(jnp.int32, (1, G), 1)
              ).astype(jnp.float32)
    h2a = jnp.concatenate([h2, jnp.ones((RT, 64), jnp.float32)], axis=1)
    contrib = lax.dot_general(onehot, h2a, (((0,), (0,)), ((), ())),
                              preferred_element_type=jnp.float32)

    @pl.when(i == 0)
    def _():
        out_ref[...] = contrib

    @pl.when(i > 0)
    def _():
        out_ref[...] += contrib

    @pl.when(i == NBLK - 1)
    def _():
        o = out_ref[...]
        out_ref[...] = o / jnp.maximum(o[:, 64:65], 1.0)


_full = lambda i: (0, 0)


def kernel(x, edge_index, batch_index, W1l, W1r, b1, W2l, W2r, b2):
    f32 = jnp.float32
    er = edge_index.reshape(2, NW, NCHUNK, NSUB, SUBW)
    bidx = batch_index.reshape(NBLK, 1, RT)
    z64 = jnp.zeros((RPT, H), f32)
    z16 = jnp.zeros((RPT, CW), f32)
    ones_h = jnp.ones((SUBW, CW), f32)
    b1t = jnp.broadcast_to(b1, (8, H))
    b2t = jnp.broadcast_to(b2, (8, OUT))

    p1, r1 = pl.pallas_call(
        _pre_body,
        grid=(NBLK,),
        in_specs=[
            pl.BlockSpec((RT, D), lambda i: (i, 0)),
            pl.BlockSpec((H, D), _full),
            pl.BlockSpec((H, D), _full),
            pl.BlockSpec((8, H), _full),
        ],
        out_specs=[pl.BlockSpec((RT, H), lambda i: (i, 0))] * 2,
        out_shape=[jax.ShapeDtypeStruct((N, H), f32)] * 2,
    )(x, W1l, W1r, b1t)

    agg1, cnt = _make_edge_agg(True)(p1, er, z64, z16, ones_h)

    a3 = pl.BlockSpec((1, RT, H), lambda i: (0, i, 0))
    a3b = pl.BlockSpec((1, RT, H), lambda i: (1, i, 0))
    c3 = pl.BlockSpec((1, RT, CW), lambda i: (0, i, 0))
    c3b = pl.BlockSpec((1, RT, CW), lambda i: (1, i, 0))
    p2, q2 = pl.pallas_call(
        _mid_body,
        grid=(NBLK,),
        in_specs=[
            a3, a3b, c3, c3b,
            pl.BlockSpec((RT, H), lambda i: (i, 0)),
            pl.BlockSpec((OUT, H), _full),
            pl.BlockSpec((OUT, H), _full),
            pl.BlockSpec((8, OUT), _full),
        ],
        out_specs=[pl.BlockSpec((RT, OUT), lambda i: (i, 0))] * 2,
        out_shape=[jax.ShapeDtypeStruct((N, OUT), f32)] * 2,
    )(agg1, agg1, cnt, cnt, r1, W2l, W2r, b2t)

    agg2, = _make_edge_agg(False)(p2, er, z64)

    pooled = pl.pallas_call(
        _pool_body,
        grid=(NBLK,),
        in_specs=[
            a3, a3b, c3, c3b,
            pl.BlockSpec((RT, OUT), lambda i: (i, 0)),
            pl.BlockSpec((1, 1, RT), lambda i: (i, 0, 0)),
        ],
        out_specs=pl.BlockSpec((G, 128), _full),
        out_shape=jax.ShapeDtypeStruct((G, 128), f32),
    )(agg2, agg2, cnt, cnt, q2, bidx)

    return pooled[:, :OUT]


# speedup vs baseline: 1.1388x; 1.0147x over previous
import functools

import jax
import jax.numpy as jnp
from jax import lax
from jax.experimental import pallas as pl
from jax.experimental.pallas import tpu as pltpu, tpu_sc as plsc

N = 10000
E = 320000
D = 128
H = 64
OUT = 64
G = 64

NC, NS = 2, 16
NW = NC * NS
CW = 16
AW = H + CW

NSUB, SUBW = 1, 250
CHUNK = NSUB * SUBW
EPT = E // NW
NCHUNK = EPT // CHUNK
DEPTH = 3
FRT = N // NS

RT = 1000
NBLK = N // RT


def _edge_agg_body(w, *refs):
    (p_hbm, er, z, agg_out, sidx, didx, rows, acc, gsem, ssem) = refs

    cid = lax.axis_index("c")
    sid = lax.axis_index("s")
    wid = sid * NC + cid

    fsl = pl.ds(sid * FRT, FRT)
    pltpu.sync_copy(z, acc.at[fsl])
    pltpu.sync_copy(er.at[0, wid], sidx)
    pltpu.sync_copy(er.at[1, wid], didx)
    plsc.subcore_barrier()

    def fire_gathers(s, b):
        for j in range(NSUB):
            pltpu.make_async_copy(p_hbm.at[sidx.at[s, j]], rows.at[b, j],
                                  gsem).start()

    def wait_gathers(b):
        for j in range(NSUB):
            pltpu.make_async_copy(p_hbm.at[sidx.at[0, j]], rows.at[b, j],
                                  gsem).wait()

    def fire_scatters(s, b):
        for j in range(NSUB):
            pltpu.make_async_copy(rows.at[b, j], acc.at[didx.at[s, j]],
                                  ssem).start(add=True)

    def drain_scatters(s, b):
        for j in range(NSUB):
            pltpu.make_async_copy(rows.at[b, j], acc.at[didx.at[s, j]],
                                  ssem).wait()

    for k in range(DEPTH - 1):
        fire_gathers(k, k)

    def chunk(s, carry):
        b = lax.rem(s, DEPTH)
        pf = s + DEPTH - 1

        @pl.when(pf < NCHUNK)
        def _():
            @pl.when(s >= 1)
            def _():
                drain_scatters(s - 1, lax.rem(s - 1, DEPTH))
            fire_gathers(pf, lax.rem(pf, DEPTH))

        wait_gathers(b)
        fire_scatters(s, b)
        return carry

    lax.fori_loop(0, NCHUNK, chunk, 0)
    for s in range(NCHUNK - DEPTH, NCHUNK):
        drain_scatters(s, s % DEPTH)
    plsc.subcore_barrier()

    pltpu.sync_copy(acc.at[fsl], agg_out.at[cid, fsl])


def _make_edge_agg(w):
    mesh = plsc.VectorSubcoreMesh(core_axis_name="c", subcore_axis_name="s",
                                  num_cores=NC, num_subcores=NS)
    return pl.kernel(
        functools.partial(_edge_agg_body, w),
        out_type=[jax.ShapeDtypeStruct((NC, N, w), jnp.float32)],
        mesh=mesh,
        scratch_types=[
            pltpu.VMEM((NCHUNK, NSUB, SUBW), jnp.int32),
            pltpu.VMEM((NCHUNK, NSUB, SUBW), jnp.int32),
            pltpu.VMEM((DEPTH, NSUB, SUBW, w), jnp.float32),
            pltpu.VMEM_SHARED((N, w), jnp.float32),
            pltpu.SemaphoreType.DMA,
            pltpu.SemaphoreType.DMA,
        ],
        compiler_params=pltpu.CompilerParams(use_tc_tiling_on_sc=False),
        name=f"edge_agg_w{w}",
    )


def _pre_body(x_ref, w1l_ref, w1r_ref, b1_ref, p1_ref, r1_ref):
    xb = x_ref[...]
    dn = (((1,), (1,)), ((), ()))
    p1 = lax.dot_general(xb, w1l_ref[...], dn,
                         preferred_element_type=jnp.float32)
    p1_ref[...] = jnp.concatenate([p1, jnp.ones((RT, CW), jnp.float32)],
                                  axis=1)
    r1_ref[...] = lax.dot_general(xb, w1r_ref[...], dn,
                                  preferred_element_type=jnp.float32) + b1_ref[0:1, :]


def _mid_body(a0_ref, a1_ref, r1_ref, w2l_ref, w2r_ref, b2_ref,
              p2_ref, q2_ref):
    a0 = a0_ref[0]
    a1 = a1_ref[0]
    inv = 1.0 / jnp.maximum(a0[:, H:H + 1] + a1[:, H:H + 1], 1.0)
    h = jnp.maximum((a0[:, :H] + a1[:, :H]) * inv + r1_ref[...], 0.0)
    dn = (((1,), (1,)), ((), ()))
    p2_ref[...] = lax.dot_general(h, w2l_ref[...], dn,
                                  preferred_element_type=jnp.float32)
    q2 = lax.dot_general(h, w2r_ref[...], dn,
                         preferred_element_type=jnp.float32) + b2_ref[0:1, :]
    q2_ref[...] = jnp.concatenate([q2, jnp.broadcast_to(inv, (RT, CW))],
                                  axis=1)


def _pool_body(a0_ref, a1_ref, q2_ref, b_ref, out_ref):
    i = pl.program_id(0)
    q2a = q2_ref[...]
    h2 = (a0_ref[0] + a1_ref[0]) * q2a[:, H:H + 1] + q2a[:, :H]
    bidx = b_ref[0, 0, :]
    onehot = (bidx[:, None] == lax.broadcasted_iota(jnp.int32, (1, G), 1)
              ).astype(jnp.float32)
    h2a = jnp.concatenate([h2, jnp.ones((RT, 64), jnp.float32)], axis=1)
    contrib = lax.dot_general(onehot, h2a, (((0,), (0,)), ((), ())),
                              preferred_element_type=jnp.float32)

    @pl.when(i == 0)
    def _():
        out_ref[...] = contrib

    @pl.when(i > 0)
    def _():
        out_ref[...] += contrib

    @pl.when(i == NBLK - 1)
    def _():
        o = out_ref[...]
        out_ref[...] = o / jnp.maximum(o[:, 64:65], 1.0)


_full = lambda i: (0, 0)


def kernel(x, edge_index, batch_index, W1l, W1r, b1, W2l, W2r, b2):
    f32 = jnp.float32
    er = edge_index.reshape(2, NW, NCHUNK, NSUB, SUBW)
    bidx = batch_index.reshape(NBLK, 1, RT)
    z80 = jnp.zeros((FRT, AW), f32)
    z64 = jnp.zeros((FRT, H), f32)
    b1t = jnp.broadcast_to(b1, (8, H))
    b2t = jnp.broadcast_to(b2, (8, OUT))

    p1, r1 = pl.pallas_call(
        _pre_body,
        grid=(NBLK,),
        in_specs=[
            pl.BlockSpec((RT, D), lambda i: (i, 0)),
            pl.BlockSpec((H, D), _full),
            pl.BlockSpec((H, D), _full),
            pl.BlockSpec((8, H), _full),
        ],
        out_specs=[pl.BlockSpec((RT, AW), lambda i: (i, 0)),
                   pl.BlockSpec((RT, H), lambda i: (i, 0))],
        out_shape=[jax.ShapeDtypeStruct((N, AW), f32),
                   jax.ShapeDtypeStruct((N, H), f32)],
    )(x, W1l, W1r, b1t)

    agg1, = _make_edge_agg(AW)(p1, er, z80)

    a80 = pl.BlockSpec((1, RT, AW), lambda i: (0, i, 0))
    a80b = pl.BlockSpec((1, RT, AW), lambda i: (1, i, 0))
    p2, q2 = pl.pallas_call(
        _mid_body,
        grid=(NBLK,),
        in_specs=[
            a80, a80b,
            pl.BlockSpec((RT, H), lambda i: (i, 0)),
            pl.BlockSpec((OUT, H), _full),
            pl.BlockSpec((OUT, H), _full),
            pl.BlockSpec((8, OUT), _full),
        ],
        out_specs=[pl.BlockSpec((RT, OUT), lambda i: (i, 0)),
                   pl.BlockSpec((RT, AW), lambda i: (i, 0))],
        out_shape=[jax.ShapeDtypeStruct((N, OUT), f32),
                   jax.ShapeDtypeStruct((N, AW), f32)],
    )(agg1, agg1, r1, W2l, W2r, b2t)

    agg2, = _make_edge_agg(H)(p2, er, z64)

    a3 = pl.BlockSpec((1, RT, H), lambda i: (0, i, 0))
    a3b = pl.BlockSpec((1, RT, H), lambda i: (1, i, 0))
    pooled = pl.pallas_call(
        _pool_body,
        grid=(NBLK,),
        in_specs=[
            a3, a3b,
            pl.BlockSpec((RT, AW), lambda i: (i, 0)),
            pl.BlockSpec((1, 1, RT), lambda i: (i, 0, 0)),
        ],
        out_specs=pl.BlockSpec((G, 128), _full),
        out_shape=jax.ShapeDtypeStruct((G, 128), f32),
    )(agg2, agg2, q2, bidx)

    return pooled[:, :OUT]


# speedup vs baseline: 1.2096x; 1.0622x over previous
import functools

import jax
import jax.numpy as jnp
from jax import lax
from jax.experimental import pallas as pl
from jax.experimental.pallas import tpu as pltpu, tpu_sc as plsc

N = 10000
E = 320000
D = 128
H = 64
OUT = 64
G = 64

NC, NS = 2, 16
NW = NC * NS
CW = 16
AW = H + CW

NSUB, SUBW = 1, 250
CHUNK = NSUB * SUBW
EPT = E // NW
NCHUNK = EPT // CHUNK
DEPTH = 3
FRT = N // NS

RT = 1000
NBLK = N // RT


def _edge_agg_body(w, *refs):
    if w == AW:
        (p_hbm, er, z, out_d, out_c, sidx, didx, rows, acc, gsem, ssem) = refs
    else:
        (p_hbm, er, z, out_d, sidx, didx, rows, acc, gsem, ssem) = refs
        out_c = None

    cid = lax.axis_index("c")
    sid = lax.axis_index("s")
    wid = sid * NC + cid

    fsl = pl.ds(sid * FRT, FRT)
    pltpu.sync_copy(z, acc.at[fsl])
    pltpu.sync_copy(er.at[0, wid], sidx)
    pltpu.sync_copy(er.at[1, wid], didx)
    plsc.subcore_barrier()

    def fire_gathers(s, b):
        for j in range(NSUB):
            pltpu.make_async_copy(p_hbm.at[sidx.at[s, j]], rows.at[b, j],
                                  gsem).start()

    def wait_gathers(b):
        for j in range(NSUB):
            pltpu.make_async_copy(p_hbm.at[sidx.at[0, j]], rows.at[b, j],
                                  gsem).wait()

    def fire_scatters(s, b):
        for j in range(NSUB):
            pltpu.make_async_copy(rows.at[b, j], acc.at[didx.at[s, j]],
                                  ssem).start(add=True)

    def drain_scatters(s, b):
        for j in range(NSUB):
            pltpu.make_async_copy(rows.at[b, j], acc.at[didx.at[s, j]],
                                  ssem).wait()

    for k in range(DEPTH - 1):
        fire_gathers(k, k)

    def chunk(s, carry):
        b = lax.rem(s, DEPTH)
        pf = s + DEPTH - 1

        @pl.when(pf < NCHUNK)
        def _():
            @pl.when(s >= 1)
            def _():
                drain_scatters(s - 1, lax.rem(s - 1, DEPTH))
            fire_gathers(pf, lax.rem(pf, DEPTH))

        wait_gathers(b)
        fire_scatters(s, b)
        return carry

    lax.fori_loop(0, NCHUNK, chunk, 0)
    for s in range(NCHUNK - DEPTH, NCHUNK):
        drain_scatters(s, s % DEPTH)
    plsc.subcore_barrier()

    if w == AW:
        pltpu.sync_copy(acc.at[fsl, pl.ds(0, H)],
                        out_d.at[fsl, pl.ds(cid * H, H)])
        pltpu.sync_copy(acc.at[fsl, pl.ds(H, CW)],
                        out_c.at[fsl, pl.ds(cid * CW, CW)])
    else:
        pltpu.sync_copy(acc.at[fsl], out_d.at[fsl, pl.ds(cid * H, H)])


def _make_edge_agg(w):
    mesh = plsc.VectorSubcoreMesh(core_axis_name="c", subcore_axis_name="s",
                                  num_cores=NC, num_subcores=NS)
    out_type = [jax.ShapeDtypeStruct((N, 2 * H), jnp.float32)]
    if w == AW:
        out_type.append(jax.ShapeDtypeStruct((N, 2 * H), jnp.float32))
    return pl.kernel(
        functools.partial(_edge_agg_body, w),
        out_type=out_type,
        mesh=mesh,
        scratch_types=[
            pltpu.VMEM((NCHUNK, NSUB, SUBW), jnp.int32),
            pltpu.VMEM((NCHUNK, NSUB, SUBW), jnp.int32),
            pltpu.VMEM((DEPTH, NSUB, SUBW, w), jnp.float32),
            pltpu.VMEM_SHARED((N, w), jnp.float32),
            pltpu.SemaphoreType.DMA,
            pltpu.SemaphoreType.DMA,
        ],
        compiler_params=pltpu.CompilerParams(use_tc_tiling_on_sc=False),
        name=f"edge_agg_w{w}",
    )


def _pre_body(x_ref, w1l_ref, w1r_ref, b1_ref, p1_ref, r1_ref):
    xb = x_ref[...]
    dn = (((1,), (1,)), ((), ()))
    p1 = lax.dot_general(xb, w1l_ref[...], dn,
                         preferred_element_type=jnp.float32)
    p1_ref[...] = jnp.concatenate([p1, jnp.ones((RT, CW), jnp.float32)],
                                  axis=1)
    r1_ref[...] = lax.dot_general(xb, w1r_ref[...], dn,
                                  preferred_element_type=jnp.float32) + b1_ref[0:1, :]


def _mid_body(d_ref, c_ref, r1_ref, w2l_ref, w2r_ref, b2_ref,
              p2_ref, q2_ref):
    d = d_ref[...]
    c = c_ref[...]
    inv = 1.0 / jnp.maximum(c[:, 0:1] + c[:, CW:CW + 1], 1.0)
    h = jnp.maximum((d[:, :H] + d[:, H:]) * inv + r1_ref[...], 0.0)
    dn = (((1,), (1,)), ((), ()))
    p2_ref[...] = lax.dot_general(h, w2l_ref[...], dn,
                                  preferred_element_type=jnp.float32)
    q2_ref[...] = lax.dot_general(h, w2r_ref[...], dn,
                                  preferred_element_type=jnp.float32) + b2_ref[0:1, :]


def _pool_body(d_ref, c_ref, q2_ref, b_ref, out_ref):
    i = pl.program_id(0)
    d = d_ref[...]
    c = c_ref[...]
    inv = 1.0 / jnp.maximum(c[:, 0:1] + c[:, CW:CW + 1], 1.0)
    h2 = (d[:, :H] + d[:, H:]) * inv + q2_ref[...]
    bidx = b_ref[0, 0, :]
    onehot = (bidx[:, None] == lax.broadcasted_iota(jnp.int32, (1, G), 1)
              ).astype(jnp.float32)
    h2a = jnp.concatenate([h2, jnp.ones((RT, 64), jnp.float32)], axis=1)
    contrib = lax.dot_general(onehot, h2a, (((0,), (0,)), ((), ())),
                              preferred_element_type=jnp.float32)

    @pl.when(i == 0)
    def _():
        out_ref[...] = contrib

    @pl.when(i > 0)
    def _():
        out_ref[...] += contrib

    @pl.when(i == NBLK - 1)
    def _():
        o = out_ref[...]
        out_ref[...] = o / jnp.maximum(o[:, 64:65], 1.0)


_full = lambda i: (0, 0)


def kernel(x, edge_index, batch_index, W1l, W1r, b1, W2l, W2r, b2):
    f32 = jnp.float32
    er = edge_index.reshape(2, NW, NCHUNK, NSUB, SUBW)
    bidx = batch_index.reshape(NBLK, 1, RT)
    z80 = jnp.zeros((FRT, AW), f32)
    z64 = jnp.zeros((FRT, H), f32)
    b1t = jnp.broadcast_to(b1, (8, H))
    b2t = jnp.broadcast_to(b2, (8, OUT))

    p1, r1 = pl.pallas_call(
        _pre_body,
        grid=(NBLK,),
        in_specs=[
            pl.BlockSpec((RT, D), lambda i: (i, 0)),
            pl.BlockSpec((H, D), _full),
            pl.BlockSpec((H, D), _full),
            pl.BlockSpec((8, H), _full),
        ],
        out_specs=[pl.BlockSpec((RT, AW), lambda i: (i, 0)),
                   pl.BlockSpec((RT, H), lambda i: (i, 0))],
        out_shape=[jax.ShapeDtypeStruct((N, AW), f32),
                   jax.ShapeDtypeStruct((N, H), f32)],
    )(x, W1l, W1r, b1t)

    d1, c1 = _make_edge_agg(AW)(p1, er, z80)

    row128 = pl.BlockSpec((RT, 2 * H), lambda i: (i, 0))
    p2, q2 = pl.pallas_call(
        _mid_body,
        grid=(NBLK,),
        in_specs=[
            row128, row128,
            pl.BlockSpec((RT, H), lambda i: (i, 0)),
            pl.BlockSpec((OUT, H), _full),
            pl.BlockSpec((OUT, H), _full),
            pl.BlockSpec((8, OUT), _full),
        ],
        out_specs=[pl.BlockSpec((RT, OUT), lambda i: (i, 0))] * 2,
        out_shape=[jax.ShapeDtypeStruct((N, OUT), f32)] * 2,
    )(d1, c1, r1, W2l, W2r, b2t)

    d2, = _make_edge_agg(H)(p2, er, z64)

    pooled = pl.pallas_call(
        _pool_body,
        grid=(NBLK,),
        in_specs=[
            row128, row128,
            pl.BlockSpec((RT, OUT), lambda i: (i, 0)),
            pl.BlockSpec((1, 1, RT), lambda i: (i, 0, 0)),
        ],
        out_specs=pl.BlockSpec((G, 128), _full),
        out_shape=jax.ShapeDtypeStruct((G, 128), f32),
    )(d2, c1, q2, bidx)

    return pooled[:, :OUT]


# speedup vs baseline: 1.2335x; 1.0197x over previous
import functools

import jax
import jax.numpy as jnp
from jax import lax
from jax.experimental import pallas as pl
from jax.experimental.pallas import tpu as pltpu, tpu_sc as plsc

N = 10000
E = 320000
D = 128
H = 64
OUT = 64
G = 64

NC, NS = 2, 16
NW = NC * NS
CW = 16
AW = H + CW

NSUB, SUBW = 1, 250
CHUNK = NSUB * SUBW
EPT = E // NW
NCHUNK = EPT // CHUNK
DEPTH = 3
FRT = N // NS

RT = 1000
NBLK = N // RT


def _edge_agg_body(w, *refs):
    if w == AW:
        (p_hbm, er, z, out_d, out_c, sidx, didx, rows, acc, gsem, ssem) = refs
    else:
        (p_hbm, er, z, out_d, sidx, didx, rows, acc, gsem, ssem) = refs
        out_c = None

    cid = lax.axis_index("c")
    sid = lax.axis_index("s")
    wid = sid * NC + cid

    fsl = pl.ds(sid * FRT, FRT)
    pltpu.sync_copy(er.at[0, wid], sidx)
    pltpu.sync_copy(er.at[1, wid], didx)

    def fire_gathers(s, b):
        for j in range(NSUB):
            pltpu.make_async_copy(p_hbm.at[sidx.at[s, j]], rows.at[b, j],
                                  gsem).start()

    def wait_gathers(b):
        for j in range(NSUB):
            pltpu.make_async_copy(p_hbm.at[sidx.at[0, j]], rows.at[b, j],
                                  gsem).wait()

    def fire_scatters(s, b):
        for j in range(NSUB):
            pltpu.make_async_copy(rows.at[b, j], acc.at[didx.at[s, j]],
                                  ssem).start(add=True)

    def drain_scatters(s, b):
        for j in range(NSUB):
            pltpu.make_async_copy(rows.at[b, j], acc.at[didx.at[s, j]],
                                  ssem).wait()

    for k in range(DEPTH - 1):
        fire_gathers(k, k)
    pltpu.sync_copy(z, acc.at[fsl])
    plsc.subcore_barrier()

    def chunk(s, carry):
        b = lax.rem(s, DEPTH)
        pf = s + DEPTH - 1

        @pl.when(pf < NCHUNK)
        def _():
            @pl.when(s >= 1)
            def _():
                drain_scatters(s - 1, lax.rem(s - 1, DEPTH))
            fire_gathers(pf, lax.rem(pf, DEPTH))

        wait_gathers(b)
        fire_scatters(s, b)
        return carry

    lax.fori_loop(0, NCHUNK, chunk, 0)
    for s in range(NCHUNK - DEPTH, NCHUNK):
        drain_scatters(s, s % DEPTH)
    plsc.subcore_barrier()

    if w == AW:
        pltpu.sync_copy(acc.at[fsl, pl.ds(0, H)],
                        out_d.at[fsl, pl.ds(cid * H, H)])
        pltpu.sync_copy(acc.at[fsl, pl.ds(H, CW)],
                        out_c.at[fsl, pl.ds(cid * CW, CW)])
    else:
        pltpu.sync_copy(acc.at[fsl], out_d.at[fsl, pl.ds(cid * H, H)])


def _make_edge_agg(w):
    mesh = plsc.VectorSubcoreMesh(core_axis_name="c", subcore_axis_name="s",
                                  num_cores=NC, num_subcores=NS)
    out_type = [jax.ShapeDtypeStruct((N, 2 * H), jnp.float32)]
    if w == AW:
        out_type.append(jax.ShapeDtypeStruct((N, 2 * H), jnp.float32))
    return pl.kernel(
        functools.partial(_edge_agg_body, w),
        out_type=out_type,
        mesh=mesh,
        scratch_types=[
            pltpu.VMEM((NCHUNK, NSUB, SUBW), jnp.int32),
            pltpu.VMEM((NCHUNK, NSUB, SUBW), jnp.int32),
            pltpu.VMEM((DEPTH, NSUB, SUBW, w), jnp.float32),
            pltpu.VMEM_SHARED((N, w), jnp.float32),
            pltpu.SemaphoreType.DMA,
            pltpu.SemaphoreType.DMA,
        ],
        compiler_params=pltpu.CompilerParams(use_tc_tiling_on_sc=False),
        name=f"edge_agg_w{w}",
    )


def _pre_body(x_ref, w1l_ref, w1r_ref, b1_ref, p1_ref, r1_ref):
    xb = x_ref[...]
    dn = (((1,), (1,)), ((), ()))
    p1 = lax.dot_general(xb, w1l_ref[...], dn,
                         preferred_element_type=jnp.float32)
    p1_ref[...] = jnp.concatenate([p1, jnp.ones((RT, CW), jnp.float32)],
                                  axis=1)
    r1_ref[...] = lax.dot_general(xb, w1r_ref[...], dn,
                                  preferred_element_type=jnp.float32) + b1_ref[0:1, :]


def _mid_body(d_ref, c_ref, r1_ref, w2l_ref, w2r_ref, b2_ref,
              p2_ref, q2_ref):
    d = d_ref[...]
    c = c_ref[...]
    inv = 1.0 / jnp.maximum(c[:, 0:1] + c[:, CW:CW + 1], 1.0)
    h = jnp.maximum((d[:, :H] + d[:, H:]) * inv + r1_ref[...], 0.0)
    dn = (((1,), (1,)), ((), ()))
    p2_ref[...] = lax.dot_general(h, w2l_ref[...], dn,
                                  preferred_element_type=jnp.float32)
    q2 = lax.dot_general(h, w2r_ref[...], dn,
                         preferred_element_type=jnp.float32) + b2_ref[0:1, :]
    q2_ref[...] = jnp.concatenate([q2, jnp.broadcast_to(inv, (RT, CW))],
                                  axis=1)


def _pool_body(d_ref, q2_ref, b_ref, out_ref):
    i = pl.program_id(0)
    d = d_ref[...]
    q2a = q2_ref[...]
    h2 = (d[:, :H] + d[:, H:]) * q2a[:, H:H + 1] + q2a[:, :H]
    bidx = b_ref[0, 0, :]
    onehot = (bidx[:, None] == lax.broadcasted_iota(jnp.int32, (1, G), 1)
              ).astype(jnp.float32)
    h2a = jnp.concatenate([h2, jnp.ones((RT, 64), jnp.float32)], axis=1)
    contrib = lax.dot_general(onehot, h2a, (((0,), (0,)), ((), ())),
                              preferred_element_type=jnp.float32)

    @pl.when(i == 0)
    def _():
        out_ref[...] = contrib

    @pl.when(i > 0)
    def _():
        out_ref[...] += contrib

    @pl.when(i == NBLK - 1)
    def _():
        o = out_ref[...]
        out_ref[...] = o / jnp.maximum(o[:, 64:65], 1.0)


_full = lambda i: (0, 0)


def kernel(x, edge_index, batch_index, W1l, W1r, b1, W2l, W2r, b2):
    f32 = jnp.float32
    er = edge_index.reshape(2, NW, NCHUNK, NSUB, SUBW)
    bidx = batch_index.reshape(NBLK, 1, RT)
    z80 = jnp.zeros((FRT, AW), f32)
    z64 = jnp.zeros((FRT, H), f32)
    b1t = jnp.broadcast_to(b1, (8, H))
    b2t = jnp.broadcast_to(b2, (8, OUT))

    p1, r1 = pl.pallas_call(
        _pre_body,
        grid=(NBLK,),
        in_specs=[
            pl.BlockSpec((RT, D), lambda i: (i, 0)),
            pl.BlockSpec((H, D), _full),
            pl.BlockSpec((H, D), _full),
            pl.BlockSpec((8, H), _full),
        ],
        out_specs=[pl.BlockSpec((RT, AW), lambda i: (i, 0)),
                   pl.BlockSpec((RT, H), lambda i: (i, 0))],
        out_shape=[jax.ShapeDtypeStruct((N, AW), f32),
                   jax.ShapeDtypeStruct((N, H), f32)],
    )(x, W1l, W1r, b1t)

    d1, c1 = _make_edge_agg(AW)(p1, er, z80)

    row128 = pl.BlockSpec((RT, 2 * H), lambda i: (i, 0))
    p2, q2 = pl.pallas_call(
        _mid_body,
        grid=(NBLK,),
        in_specs=[
            row128, row128,
            pl.BlockSpec((RT, H), lambda i: (i, 0)),
            pl.BlockSpec((OUT, H), _full),
            pl.BlockSpec((OUT, H), _full),
            pl.BlockSpec((8, OUT), _full),
        ],
        out_specs=[pl.BlockSpec((RT, OUT), lambda i: (i, 0)),
                   pl.BlockSpec((RT, AW), lambda i: (i, 0))],
        out_shape=[jax.ShapeDtypeStruct((N, OUT), f32),
                   jax.ShapeDtypeStruct((N, AW), f32)],
    )(d1, c1, r1, W2l, W2r, b2t)

    d2, = _make_edge_agg(H)(p2, er, z64)

    pooled = pl.pallas_call(
        _pool_body,
        grid=(NBLK,),
        in_specs=[
            row128,
            pl.BlockSpec((RT, AW), lambda i: (i, 0)),
            pl.BlockSpec((1, 1, RT), lambda i: (i, 0, 0)),
        ],
        out_specs=pl.BlockSpec((G, 128), _full),
        out_shape=jax.ShapeDtypeStruct((G, 128), f32),
    )(d2, q2, bidx)

    return pooled[:, :OUT]
